# baseline (speedup 1.0000x reference)
import functools
import jax
import jax.numpy as jnp
from jax import lax
from jax.experimental import pallas as pl
from jax.experimental.pallas import tpu as pltpu
from jax.experimental.pallas import tpu_sc as plsc

N = 10000
E = 160000
S = 1024
DIN = 2050
D = 256
H = 128
NP = 10240
TPT = NP // 16
EPT = E // 16
G = 128
NCH = 79
EPP = NCH * G
NA = 10016
MB = 400
SACC = 1040

@functools.lru_cache(maxsize=None)
def _mesh():
    return plsc.VectorSubcoreMesh(core_axis_name="c", subcore_axis_name="s")

_f32 = jnp.float32
_i32 = jnp.int32


SP = 1152


def _counts_body(d0, d1, d2, d3, d4, d5, x0, x1, x2, x3, x4, x5, z1,
                 oe0, oe1, oe2, oe3, oe4, oe5,
                 os0, os1, os2, os3, os4, os5,
                 ebuf, xbuf, priv, privs):
    c = lax.axis_index("c")
    s = lax.axis_index("s")
    ones = jnp.full((16,), 1.0, _f32)
    dsets = ((d0, d1, d2), (d3, d4, d5))
    xsets = ((x0, x1, x2), (x3, x4, x5))
    eouts = ((oe0, oe1, oe2), (oe3, oe4, oe5))
    souts = ((os0, os1, os2), (os3, os4, os5))

    for k in range(3):
        for core in range(2):
            @pl.when(c == core)
            def _(dref=dsets[core][k], oref=eouts[core][k]):
                pltpu.sync_copy(z1.at[pl.ds(0, NP)], priv)
                pltpu.sync_copy(dref.at[pl.ds(s * EPT, EPT)], ebuf)

                def it(i, carry):
                    ii = ebuf[pl.ds(i * 16, 16)]
                    plsc.addupdate_scatter(priv, [ii], ones)
                    return carry

                lax.fori_loop(0, EPT // 16, it, 0)
                pltpu.sync_copy(priv, oref.at[pl.ds(s * NP, NP)])

    for k in range(3):
        for core in range(2):
            @pl.when(c == core)
            def _(xref=xsets[core][k], oref=souts[core][k]):
                pltpu.sync_copy(z1.at[pl.ds(0, SP)], privs)
                pltpu.sync_copy(xref.at[pl.ds(s * TPT, TPT)], xbuf)

                def it(i, carry):
                    ii = xbuf[pl.ds(i * 16, 16)]
                    plsc.addupdate_scatter(privs, [ii], ones)
                    return carry

                lax.fori_loop(0, TPT // 16, it, 0)
                pltpu.sync_copy(privs, oref.at[pl.ds(s * SP, SP)])


@functools.lru_cache(maxsize=None)
def _counts_kernel():
    return pl.kernel(
        _counts_body, mesh=_mesh(),
        compiler_params=pltpu.CompilerParams(needs_layout_passes=False),
        out_type=[jax.ShapeDtypeStruct((16 * NP,), _f32) for _ in range(6)]
        + [jax.ShapeDtypeStruct((16 * SP,), _f32) for _ in range(6)],
        scratch_types=[
            pltpu.VMEM((EPT,), _i32),
            pltpu.VMEM((TPT,), _i32),
            pltpu.VMEM((NP,), _f32),
            pltpu.VMEM((SP,), _f32),
        ],
    )


def _csum_body(*refs):
    ins = refs[:12]
    outs = refs[12:]
    for i in range(12):
        outs[i][...] = jnp.sum(ins[i][...], axis=0)


def _csum(couts):
    ins = [cc.reshape(16, NP) for cc in couts[:6]]
    ins += [cc.reshape(16, SP) for cc in couts[6:]]
    return pl.pallas_call(
        _csum_body,
        out_shape=[jax.ShapeDtypeStruct((NP,), _f32) for _ in range(6)]
        + [jax.ShapeDtypeStruct((SP,), _f32) for _ in range(6)],
    )(*ins)


def _gcn_body(epk, ha, hb, z, oa, ob,
              ib0, ib1, r0, r1, wb, acc,
              gs0, gs1, ss0, ss1):
    c = lax.axis_index("c")
    s = lax.axis_index("s")
    pltpu.sync_copy(z.at[pl.ds(0, 104)], wb)
    for j in range(6):
        pltpu.sync_copy(wb, acc.at[pl.ds(s * 624 + j * 104, 104)])

    @pl.when(s == 15)
    def _():
        pltpu.sync_copy(wb.at[pl.ds(0, 32)], acc.at[pl.ds(9984, 32)])

    plsc.subcore_barrier()
    ibufs = (ib0, ib1)
    rws = (r0, r1)
    gsems = (gs0, gs1)
    ssems = (ss0, ss1)

    def run(href):
        pltpu.sync_copy(epk.at[s, 0], ib0)
        pltpu.async_copy(href.at[ib0.at[0]], r0, gs0)

        def outer(k, carry):
            for b in (0, 1):
                i = 2 * k + b
                nb = 1 - b

                @pl.when(i + 1 < NCH)
                def _():
                    @pl.when(i >= 1)
                    def _():
                        pltpu.make_async_copy(
                            rws[nb], acc.at[ibufs[nb].at[1]],
                            ssems[nb]).wait()

                    pltpu.sync_copy(epk.at[s, i + 1], ibufs[nb])
                    pltpu.async_copy(href.at[ibufs[nb].at[0]], rws[nb],
                                     gsems[nb])

                pltpu.make_async_copy(href.at[ibufs[b].at[0]], rws[b],
                                      gsems[b]).wait()
                pltpu.async_copy(rws[b], acc.at[ibufs[b].at[1]], ssems[b],
                                 add=True)
            return carry

        lax.fori_loop(0, NCH // 2, outer, 0)
        pltpu.make_async_copy(href.at[ib0.at[0]], r0, gs0).wait()
        pltpu.async_copy(r0, acc.at[ib0.at[1]], ss0, add=True)
        pltpu.make_async_copy(r1, acc.at[ib1.at[1]], ss1).wait()
        pltpu.make_async_copy(r0, acc.at[ib0.at[1]], ss0).wait()

    for core, href in ((0, ha), (1, hb)):
        @pl.when(c == core)
        def _(href=href):
            run(href)

    plsc.subcore_barrier()
    for j in range(6):
        pltpu.sync_copy(acc.at[pl.ds(s * 624 + j * 104, 104)], wb)
        pl.when(c == 0)(lambda j=j: pltpu.sync_copy(
            wb, oa.at[pl.ds(s * 624 + j * 104, 104)]))
        pl.when(c == 1)(lambda j=j: pltpu.sync_copy(
            wb, ob.at[pl.ds(s * 624 + j * 104, 104)]))

    @pl.when(s == 15)
    def _():
        pltpu.sync_copy(acc.at[pl.ds(9984, 16)], r0.at[pl.ds(0, 16)])
        pl.when(c == 0)(lambda: pltpu.sync_copy(
            r0.at[pl.ds(0, 16)], oa.at[pl.ds(9984, 16)]))
        pl.when(c == 1)(lambda: pltpu.sync_copy(
            r0.at[pl.ds(0, 16)], ob.at[pl.ds(9984, 16)]))


@functools.lru_cache(maxsize=None)
def _gcn_kernel():
    return pl.kernel(
        _gcn_body, mesh=_mesh(),
        compiler_params=pltpu.CompilerParams(needs_layout_passes=False),
        out_type=[jax.ShapeDtypeStruct((N, H), _f32),
                  jax.ShapeDtypeStruct((N, H), _f32)],
        scratch_types=[
            pltpu.VMEM((2, G), _i32),
            pltpu.VMEM((2, G), _i32),
            pltpu.VMEM((G, 128), _f32),
            pltpu.VMEM((G, 128), _f32),
            pltpu.VMEM((104, 128), _f32),
            pltpu.VMEM_SHARED((NA, 128), _f32),
            pltpu.SemaphoreType.DMA,
            pltpu.SemaphoreType.DMA,
            pltpu.SemaphoreType.DMA,
            pltpu.SemaphoreType.DMA,
        ],
    )


def _ssum_body(h0a, h0b, h1a, h1b, h2a, h2b,
               i0, i1, i2, i3, i4, i5, z,
               o0a, o0b, o1a, o1b, o2a, o2b,
               o3a, o3b, o4a, o4b, o5a, o5b,
               ibufA, ibufB, rows, wb,
               a0, a1, a2, a3, a4, a5):
    c = lax.axis_index("c")
    s = lax.axis_index("s")
    accs = (a0, a1, a2, a3, a4, a5)
    pltpu.sync_copy(z.at[pl.ds(0, 64)], wb)
    for a in accs:
        pltpu.sync_copy(wb, a.at[pl.ds(s * 64, 64)])
    plsc.subcore_barrier()

    hs = ((h0a, h0b), (h1a, h1b), (h2a, h2b))
    idxs = ((i0, i1), (i2, i3), (i4, i5))
    for p in range(3):
        for core in range(2):
            @pl.when(c == core)
            def _(href=hs[p][core], p=p):
                accA = accs[2 * p]
                accB = accs[2 * p + 1]
                iA = idxs[p][0]
                iB = idxs[p][1]

                def it(i, carry):
                    off = s * TPT + i * 80
                    pltpu.sync_copy(iA.at[pl.ds(off, 80)], ibufA)
                    pltpu.sync_copy(iB.at[pl.ds(off, 80)], ibufB)
                    pltpu.sync_copy(href.at[pl.ds(off, 80)], rows)
                    pltpu.sync_copy(rows, accA.at[ibufA], add=True)
                    pltpu.sync_copy(rows, accB.at[ibufB], add=True)
                    return carry

                lax.fori_loop(0, TPT // 80, it, 0)

    plsc.subcore_barrier()
    outs = ((o0a, o0b), (o1a, o1b), (o2a, o2b),
            (o3a, o3b), (o4a, o4b), (o5a, o5b))
    for k in range(6):
        pltpu.sync_copy(accs[k].at[pl.ds(s * 64, 64)], wb)
        for core in range(2):
            pl.when(c == core)(
                lambda oref=outs[k][core]: pltpu.sync_copy(
                    wb, oref.at[pl.ds(s * 64, 64)]))


@functools.lru_cache(maxsize=None)
def _ssum_kernel():
    return pl.kernel(
        _ssum_body, mesh=_mesh(),
        compiler_params=pltpu.CompilerParams(needs_layout_passes=False),
        out_type=[jax.ShapeDtypeStruct((S, H), _f32) for _ in range(12)],
        scratch_types=[
            pltpu.VMEM((80,), _i32),
            pltpu.VMEM((80,), _i32),
            pltpu.VMEM((80, 128), _f32),
            pltpu.VMEM((64, 128), _f32),
        ] + [pltpu.VMEM_SHARED((SACC, 128), _f32) for _ in range(6)],
    )


def _mm1_body(x_ref, w1_ref, w2_ref, c1_ref, c2_ref,
              o1a, o1b, o2a, o2b):
    x = x_ref[...]
    h1 = jnp.dot(x, w1_ref[...], preferred_element_type=_f32)
    h2 = jnp.dot(x, w2_ref[...], preferred_element_type=_f32)
    d1 = lax.rsqrt(c1_ref[...] + 1.0)
    d2 = lax.rsqrt(c2_ref[...] + 1.0)
    h1 = h1 * d1
    h2 = h2 * d2
    o1a[...] = h1[:, :H]
    o1b[...] = h1[:, H:]
    o2a[...] = h2[:, :H]
    o2b[...] = h2[:, H:]


def _mm1(x, w1, w2, cnt1, cnt2):
    out = jax.ShapeDtypeStruct((N, H), _f32)
    return pl.pallas_call(
        _mm1_body,
        grid=(N // MB,),
        in_specs=[
            pl.BlockSpec((MB, DIN), lambda i: (i, 0)),
            pl.BlockSpec((DIN, D), lambda i: (0, 0)),
            pl.BlockSpec((DIN, D), lambda i: (0, 0)),
            pl.BlockSpec((MB, 1), lambda i: (i, 0)),
            pl.BlockSpec((MB, 1), lambda i: (i, 0)),
        ],
        out_specs=[pl.BlockSpec((MB, H), lambda i: (i, 0))] * 4,
        out_shape=[out, out, out, out],
    )(x, w1, w2, cnt1, cnt2)


def _mid_body(s1a, s1b, s2a, s2b, g1a, g1b, g2a, g2b, c1, c2,
              bc1, bc2, mw1, mb1, mw2, mb2, wn1, wn2,
              o1a, o1b, o2a, o2b):
    d1 = lax.rsqrt(c1[...] + 1.0)
    d2 = lax.rsqrt(c2[...] + 1.0)
    s1 = jnp.concatenate([s1a[...], s1b[...]], axis=1)
    s2 = jnp.concatenate([s2a[...], s2b[...]], axis=1)
    g1 = jnp.concatenate([g1a[...], g1b[...]], axis=1)
    g2 = jnp.concatenate([g2a[...], g2b[...]], axis=1)
    h1 = jnp.maximum(d1 * (s1 + g1) + bc1[...], 0.0)
    h2 = jnp.maximum(d2 * (s2 + g2) + bc2[...], 0.0)
    hcat = jnp.concatenate([h1, h2], axis=1)
    t = jnp.maximum(jnp.dot(hcat, mw1[...], preferred_element_type=_f32)
                    + mb1[...], 0.0)
    m = jnp.dot(t, mw2[...], preferred_element_type=_f32) + mb2[...]
    n1 = jnp.dot(m, wn1[...], preferred_element_type=_f32) * d1
    n2 = jnp.dot(m, wn2[...], preferred_element_type=_f32) * d2
    o1a[...] = n1[:, :H]
    o1b[...] = n1[:, H:]
    o2a[...] = n2[:, :H]
    o2b[...] = n2[:, H:]


def _mid(scats, gs, cnt1, cnt2, bc1, bc2, mw1, mb1, mw2, mb2, wn1, wn2):
    out = jax.ShapeDtypeStruct((N, H), _f32)
    blk = pl.BlockSpec((MB, H), lambda i: (i, 0))
    full = lambda shp: pl.BlockSpec(shp, lambda i: (0, 0))
    return pl.pallas_call(
        _mid_body,
        grid=(N // MB,),
        in_specs=[blk] * 8 + [
            pl.BlockSpec((MB, 1), lambda i: (i, 0)),
            pl.BlockSpec((MB, 1), lambda i: (i, 0)),
            full((1, D)), full((1, D)),
            full((2 * D, D)), full((1, D)), full((D, D)), full((1, D)),
            full((D, D)), full((D, D)),
        ],
        out_specs=[blk] * 4,
        out_shape=[out, out, out, out],
    )(*scats, *gs, cnt1, cnt2, bc1, bc2, mw1, mb1, mw2, mb2, wn1, wn2)


def _tail_body(s1a, s1b, s2a, s2b, g1a, g1b, g2a, g2b, c1, c2,
               bc1, bc2, mw1, mb1, mw2, mb2, oa, ob):
    d1 = lax.rsqrt(c1[...] + 1.0)
    d2 = lax.rsqrt(c2[...] + 1.0)
    s1 = jnp.concatenate([s1a[...], s1b[...]], axis=1)
    s2 = jnp.concatenate([s2a[...], s2b[...]], axis=1)
    g1 = jnp.concatenate([g1a[...], g1b[...]], axis=1)
    g2 = jnp.concatenate([g2a[...], g2b[...]], axis=1)
    h1 = jnp.maximum(d1 * (s1 + g1) + bc1[...], 0.0)
    h2 = jnp.maximum(d2 * (s2 + g2) + bc2[...], 0.0)
    hcat = jnp.concatenate([h1, h2], axis=1)
    t = jnp.maximum(jnp.dot(hcat, mw1[...], preferred_element_type=_f32)
                    + mb1[...], 0.0)
    m = jnp.dot(t, mw2[...], preferred_element_type=_f32) + mb2[...]
    oa[...] = m[:, :H]
    ob[...] = m[:, H:]


def _tail(scats, gs, cnt1, cnt2, bc1, bc2, mw1, mb1, mw2, mb2):
    out = jax.ShapeDtypeStruct((N, H), _f32)
    blk = pl.BlockSpec((MB, H), lambda i: (i, 0))
    full = lambda shp: pl.BlockSpec(shp, lambda i: (0, 0))
    return pl.pallas_call(
        _tail_body,
        grid=(N // MB,),
        in_specs=[blk] * 8 + [
            pl.BlockSpec((MB, 1), lambda i: (i, 0)),
            pl.BlockSpec((MB, 1), lambda i: (i, 0)),
            full((1, D)), full((1, D)),
            full((2 * D, D)), full((1, D)), full((D, D)), full((1, D)),
        ],
        out_specs=[blk, blk],
        out_shape=[out, out],
    )(*scats, *gs, cnt1, cnt2, bc1, bc2, mw1, mb1, mw2, mb2)


def _final_body(t0a, t0b, t1a, t1b, t2a, t2b, t3a, t3b, t4a, t4b, t5a, t5b,
                c0, c1, c2, c3, c4, c5,
                w31, b31, w32, b32, wf1, bf1, wf2, bf2, out_ref):
    def mean(ta, tb, cnt):
        t = jnp.concatenate([ta[...], tb[...]], axis=1)
        return t / jnp.maximum(cnt[...], 1.0)

    x1 = mean(t0a, t0b, c0)
    x2 = mean(t1a, t1b, c1)
    xo1 = mean(t2a, t2b, c2)
    xo2 = mean(t3a, t3b, c3)
    xi1 = mean(t4a, t4b, c4)
    xi2 = mean(t5a, t5b, c5)

    def mlp3(a, b):
        hh = jnp.concatenate([a, b], axis=1)
        t = jnp.maximum(jnp.dot(hh, w31[...], preferred_element_type=_f32)
                        + b31[...], 0.0)
        return jnp.dot(t, w32[...], preferred_element_type=_f32) + b32[...]

    x_ = mlp3(x1, x2)
    xout = mlp3(xo1, xo2)
    xin = mlp3(xi1, xi2)
    xin = jnp.where(c4[...] > 0.0, xin, x_)
    xout = jnp.where(c2[...] > 0.0, xout, x_)

    hcat = jnp.concatenate([x_, xin, xout], axis=1)
    t = jnp.maximum(jnp.dot(hcat, wf1[...], preferred_element_type=_f32)
                    + bf1[...], 0.0)
    o = jnp.dot(t, wf2[...], preferred_element_type=_f32) + bf2[...]
    mx = jnp.max(o, axis=1, keepdims=True)
    e = jnp.exp(o - mx)
    lse = jnp.log(jnp.sum(e, axis=1, keepdims=True))
    out_ref[...] = o - mx - lse


def _final(tots, cnts, w31, b31, w32, b32, wf1, bf1, wf2, bf2):
    return pl.pallas_call(
        _final_body,
        out_shape=jax.ShapeDtypeStruct((S, D), _f32),
    )(*tots, *cnts, w31, b31, w32, b32, wf1, bf1, wf2, bf2)


def kernel(x, x_out, x_in,
           edge_index_1, edge_index_2, edge_index_out_1, edge_index_out_2,
           edge_index_in_1, edge_index_in_2,
           index_1, index_2, index_out_1, index_out_2, index_in_1,
           index_in_2,
           W_c11, b_c11, W_c12, b_c12, W_c21, b_c21, W_c22, b_c22,
           m1_W1, m1_b1, m1_W2, m1_b2,
           m2_W1, m2_b1, m2_W2, m2_b2,
           m3_W1, m3_b1, m3_W2, m3_b2,
           mlp_W1, mlp_b1, mlp_W2, mlp_b2):
    edges = (edge_index_1, edge_index_2, edge_index_out_1, edge_index_out_2,
             edge_index_in_1, edge_index_in_2)
    idxs = (index_1, index_2, index_out_1, index_out_2, index_in_1,
            index_in_2)

    def pack_edges(e):
        sr = jnp.pad(e[0].reshape(16, EPT), ((0, 0), (0, EPP - EPT)))
        dr = jnp.pad(e[1].reshape(16, EPT), ((0, 0), (0, EPP - EPT)),
                     constant_values=N)
        return jnp.stack([sr.reshape(16, NCH, G), dr.reshape(16, NCH, G)],
                         axis=2)

    epks = [pack_edges(e) for e in edges]
    dsts = [e[1] for e in edges]
    idx_pad = [jnp.pad(ix, (0, NP - N), constant_values=S) for ix in idxs]
    zeros = jnp.zeros((625, 128), _f32)
    zeros1 = jnp.zeros((NP,), _f32)

    couts = _counts_kernel()(*dsts, *idx_pad, zeros1)
    sums = _csum(couts)
    ecnt = [sums[j][:N].reshape(N, 1) for j in range(6)]
    scnt = [sums[6 + j][:S].reshape(S, 1) for j in range(6)]

    b_c11r = b_c11.reshape(1, D)
    b_c12r = b_c12.reshape(1, D)
    b_c21r = b_c21.reshape(1, D)
    b_c22r = b_c22.reshape(1, D)
    m1_b1r = m1_b1.reshape(1, D)
    m1_b2r = m1_b2.reshape(1, D)
    m2_b1r = m2_b1.reshape(1, D)
    m2_b2r = m2_b2.reshape(1, D)

    def branch(xb, e1, e2):
        c1, c2 = ecnt[e1], ecnt[e2]
        g1a, g1b, g2a, g2b = _mm1(xb, W_c11, W_c12, c1, c2)
        s1a, s1b = _gcn_kernel()(epks[e1], g1a, g1b, zeros)
        s2a, s2b = _gcn_kernel()(epks[e2], g2a, g2b, zeros)
        n1a, n1b, n2a, n2b = _mid(
            (s1a, s1b, s2a, s2b), (g1a, g1b, g2a, g2b), c1, c2,
            b_c11r, b_c12r, m1_W1, m1_b1r, m1_W2, m1_b2r, W_c21, W_c22)
        s1a, s1b = _gcn_kernel()(epks[e1], n1a, n1b, zeros)
        s2a, s2b = _gcn_kernel()(epks[e2], n2a, n2b, zeros)
        return _tail((s1a, s1b, s2a, s2b), (n1a, n1b, n2a, n2b), c1, c2,
                     b_c21r, b_c22r, m2_W1, m2_b1r, m2_W2, m2_b2r)

    hoa, hob = branch(x, 0, 1)
    houta, houtb = branch(x_out, 2, 3)
    hina, hinb = branch(x_in, 4, 5)

    pad2 = lambda a: jnp.pad(a, ((0, NP - N), (0, 0)))
    tots = _ssum_kernel()(
        pad2(hoa), pad2(hob), pad2(houta), pad2(houtb),
        pad2(hina), pad2(hinb),
        idx_pad[0], idx_pad[1], idx_pad[2], idx_pad[3], idx_pad[4],
        idx_pad[5], zeros)

    return _final(tots, scnt,
                  m3_W1, m3_b1.reshape(1, D), m3_W2, m3_b2.reshape(1, D),
                  mlp_W1, mlp_b1.reshape(1, 2 * D),
                  mlp_W2, mlp_b2.reshape(1, D))


# speedup vs baseline: 1.3520x; 1.3520x over previous
import functools
import jax
import jax.numpy as jnp
from jax import lax
from jax.experimental import pallas as pl
from jax.experimental.pallas import tpu as pltpu
from jax.experimental.pallas import tpu_sc as plsc

N = 10000
E = 160000
S = 1024
DIN = 2050
D = 256
H = 128
NP = 10240
TPT = NP // 16
EPT = E // 16
G = 128
NCH = EPT // G
GT = EPT - NCH * G
MB = 400
SACC = 1040

@functools.lru_cache(maxsize=None)
def _mesh():
    return plsc.VectorSubcoreMesh(core_axis_name="c", subcore_axis_name="s")

_f32 = jnp.float32
_i32 = jnp.int32


SP = 1152


def _counts_body(d0, d1, d2, d3, d4, d5, x0, x1, x2, x3, x4, x5, z1,
                 oe0, oe1, oe2, oe3, oe4, oe5,
                 os0, os1, os2, os3, os4, os5,
                 ebuf, xbuf, priv, privs):
    c = lax.axis_index("c")
    s = lax.axis_index("s")
    ones = jnp.full((16,), 1.0, _f32)
    dsets = ((d0, d1, d2), (d3, d4, d5))
    xsets = ((x0, x1, x2), (x3, x4, x5))
    eouts = ((oe0, oe1, oe2), (oe3, oe4, oe5))
    souts = ((os0, os1, os2), (os3, os4, os5))

    for k in range(3):
        for core in range(2):
            @pl.when(c == core)
            def _(dref=dsets[core][k], oref=eouts[core][k]):
                pltpu.sync_copy(z1.at[pl.ds(0, NP)], priv)
                pltpu.sync_copy(dref.at[pl.ds(s * EPT, EPT)], ebuf)

                def it(i, carry):
                    ii = ebuf[pl.ds(i * 16, 16)]
                    plsc.addupdate_scatter(priv, [ii], ones)
                    return carry

                lax.fori_loop(0, EPT // 16, it, 0)
                pltpu.sync_copy(priv, oref.at[pl.ds(s * NP, NP)])

    for k in range(3):
        for core in range(2):
            @pl.when(c == core)
            def _(xref=xsets[core][k], oref=souts[core][k]):
                pltpu.sync_copy(z1.at[pl.ds(0, SP)], privs)
                pltpu.sync_copy(xref.at[pl.ds(s * TPT, TPT)], xbuf)

                def it(i, carry):
                    ii = xbuf[pl.ds(i * 16, 16)]
                    plsc.addupdate_scatter(privs, [ii], ones)
                    return carry

                lax.fori_loop(0, TPT // 16, it, 0)
                pltpu.sync_copy(privs, oref.at[pl.ds(s * SP, SP)])


@functools.lru_cache(maxsize=None)
def _counts_kernel():
    return pl.kernel(
        _counts_body, mesh=_mesh(),
        compiler_params=pltpu.CompilerParams(needs_layout_passes=False),
        out_type=[jax.ShapeDtypeStruct((16 * NP,), _f32) for _ in range(6)]
        + [jax.ShapeDtypeStruct((16 * SP,), _f32) for _ in range(6)],
        scratch_types=[
            pltpu.VMEM((EPT,), _i32),
            pltpu.VMEM((TPT,), _i32),
            pltpu.VMEM((NP,), _f32),
            pltpu.VMEM((SP,), _f32),
        ],
    )


def _csum_body(*refs):
    ins = refs[:12]
    outs = refs[12:]
    for i in range(12):
        outs[i][...] = jnp.sum(ins[i][...], axis=0)


def _csum(couts):
    ins = [cc.reshape(16, NP) for cc in couts[:6]]
    ins += [cc.reshape(16, SP) for cc in couts[6:]]
    return pl.pallas_call(
        _csum_body,
        out_shape=[jax.ShapeDtypeStruct((NP,), _f32) for _ in range(6)]
        + [jax.ShapeDtypeStruct((SP,), _f32) for _ in range(6)],
    )(*ins)


def _gcn_body(src, dst, ha, hb, z, oa, ob,
              sb0, sb1, sb2, sb3, db0, db1, db2, db3, st, dt,
              r0, r1, wb, acc,
              gs0, gs1, ss0, ss1, is0, is1, is2, is3):
    c = lax.axis_index("c")
    s = lax.axis_index("s")
    pltpu.sync_copy(z.at[pl.ds(0, 104)], wb)
    for j in range(6):
        pltpu.sync_copy(wb, acc.at[pl.ds(s * 624 + j * 104, 104)])

    @pl.when(s == 15)
    def _():
        pltpu.sync_copy(wb.at[pl.ds(0, 16)], acc.at[pl.ds(9984, 16)])

    plsc.subcore_barrier()
    sbufs = (sb0, sb1, sb2, sb3)
    dbufs = (db0, db1, db2, db3)
    rws = (r0, r1)
    gsems = (gs0, gs1)
    ssems = (ss0, ss1)
    isems = (is0, is1, is2, is3)
    base = s * EPT

    def ld_idx(i, j):
        off = base + i * G
        pltpu.async_copy(src.at[pl.ds(off, G)], sbufs[j], isems[j])
        pltpu.async_copy(dst.at[pl.ds(off, G)], dbufs[j], isems[j])

    def wait_idx(i, j):
        off = base + i * G
        pltpu.make_async_copy(src.at[pl.ds(off, G)], sbufs[j],
                              isems[j]).wait()
        pltpu.make_async_copy(dst.at[pl.ds(off, G)], dbufs[j],
                              isems[j]).wait()

    def run(href):
        pltpu.sync_copy(src.at[pl.ds(base, G)], sb0)
        pltpu.sync_copy(dst.at[pl.ds(base, G)], db0)
        ld_idx(1, 1)
        ld_idx(2, 2)
        pltpu.async_copy(href.at[sb0], r0, gs0)

        def sub(i, b, j):
            nb = 1 - b
            pltpu.make_async_copy(href.at[sbufs[j]], rws[b],
                                  gsems[b]).wait()
            pltpu.async_copy(rws[b], acc.at[dbufs[j]], ssems[b], add=True)

            @pl.when(i + 1 < NCH)
            def _():
                wait_idx(i + 1, (j + 1) % 4)

            @pl.when(i >= 1)
            def _():
                pltpu.make_async_copy(
                    rws[nb], acc.at[dbufs[(j + 3) % 4]],
                    ssems[nb]).wait()

            @pl.when(i + 1 < NCH)
            def _():
                pltpu.async_copy(href.at[sbufs[(j + 1) % 4]], rws[nb],
                                 gsems[nb])

            @pl.when(i + 3 < NCH)
            def _():
                ld_idx(i + 3, (j + 3) % 4)

        def outer(k, carry):
            for u in range(4):
                sub(4 * k + u, u % 2, u)
            return carry

        lax.fori_loop(0, NCH // 4, outer, 0)
        for i in range(NCH - NCH % 4, NCH):
            sub(i, i % 2, i % 4)
        pltpu.make_async_copy(rws[(NCH - 1) % 2],
                              acc.at[dbufs[(NCH - 1) % 4]],
                              ssems[(NCH - 1) % 2]).wait()
        pltpu.sync_copy(src.at[pl.ds(base + NCH * G, GT)], st)
        pltpu.sync_copy(dst.at[pl.ds(base + NCH * G, GT)], dt)
        pltpu.async_copy(href.at[st], r0.at[pl.ds(0, GT)], gs0).wait()
        pltpu.sync_copy(r0.at[pl.ds(0, GT)], acc.at[dt], add=True)

    for core, href in ((0, ha), (1, hb)):
        @pl.when(c == core)
        def _(href=href):
            run(href)

    plsc.subcore_barrier()
    for j in range(6):
        pltpu.sync_copy(acc.at[pl.ds(s * 624 + j * 104, 104)], wb)
        pl.when(c == 0)(lambda j=j: pltpu.sync_copy(
            wb, oa.at[pl.ds(s * 624 + j * 104, 104)]))
        pl.when(c == 1)(lambda j=j: pltpu.sync_copy(
            wb, ob.at[pl.ds(s * 624 + j * 104, 104)]))

    @pl.when(s == 15)
    def _():
        pltpu.sync_copy(acc.at[pl.ds(9984, 16)], r0.at[pl.ds(0, 16)])
        pl.when(c == 0)(lambda: pltpu.sync_copy(
            r0.at[pl.ds(0, 16)], oa.at[pl.ds(9984, 16)]))
        pl.when(c == 1)(lambda: pltpu.sync_copy(
            r0.at[pl.ds(0, 16)], ob.at[pl.ds(9984, 16)]))


@functools.lru_cache(maxsize=None)
def _gcn_kernel():
    return pl.kernel(
        _gcn_body, mesh=_mesh(),
        compiler_params=pltpu.CompilerParams(needs_layout_passes=False),
        out_type=[jax.ShapeDtypeStruct((N, H), _f32),
                  jax.ShapeDtypeStruct((N, H), _f32)],
        scratch_types=(
            [pltpu.VMEM((G,), _i32) for _ in range(8)]
            + [pltpu.VMEM((GT,), _i32) for _ in range(2)]
            + [
                pltpu.VMEM((G, 128), _f32),
                pltpu.VMEM((G, 128), _f32),
                pltpu.VMEM((104, 128), _f32),
                pltpu.VMEM_SHARED((N, 128), _f32),
            ]
            + [pltpu.SemaphoreType.DMA for _ in range(8)]
        ),
    )


def _ssum_body(h0a, h0b, h1a, h1b, h2a, h2b,
               i0, i1, i2, i3, i4, i5, z,
               o0a, o0b, o1a, o1b, o2a, o2b,
               o3a, o3b, o4a, o4b, o5a, o5b,
               ibufA, ibufB, rows, wb,
               a0, a1, a2, a3, a4, a5):
    c = lax.axis_index("c")
    s = lax.axis_index("s")
    accs = (a0, a1, a2, a3, a4, a5)
    pltpu.sync_copy(z.at[pl.ds(0, 64)], wb)
    for a in accs:
        pltpu.sync_copy(wb, a.at[pl.ds(s * 64, 64)])
    plsc.subcore_barrier()

    hs = ((h0a, h0b), (h1a, h1b), (h2a, h2b))
    idxs = ((i0, i1), (i2, i3), (i4, i5))
    for p in range(3):
        for core in range(2):
            @pl.when(c == core)
            def _(href=hs[p][core], p=p):
                accA = accs[2 * p]
                accB = accs[2 * p + 1]
                iA = idxs[p][0]
                iB = idxs[p][1]

                def it(i, carry):
                    off = s * TPT + i * 80
                    pltpu.sync_copy(iA.at[pl.ds(off, 80)], ibufA)
                    pltpu.sync_copy(iB.at[pl.ds(off, 80)], ibufB)
                    pltpu.sync_copy(href.at[pl.ds(off, 80)], rows)
                    pltpu.sync_copy(rows, accA.at[ibufA], add=True)
                    pltpu.sync_copy(rows, accB.at[ibufB], add=True)
                    return carry

                lax.fori_loop(0, TPT // 80, it, 0)

    plsc.subcore_barrier()
    outs = ((o0a, o0b), (o1a, o1b), (o2a, o2b),
            (o3a, o3b), (o4a, o4b), (o5a, o5b))
    for k in range(6):
        pltpu.sync_copy(accs[k].at[pl.ds(s * 64, 64)], wb)
        for core in range(2):
            pl.when(c == core)(
                lambda oref=outs[k][core]: pltpu.sync_copy(
                    wb, oref.at[pl.ds(s * 64, 64)]))


@functools.lru_cache(maxsize=None)
def _ssum_kernel():
    return pl.kernel(
        _ssum_body, mesh=_mesh(),
        compiler_params=pltpu.CompilerParams(needs_layout_passes=False),
        out_type=[jax.ShapeDtypeStruct((S, H), _f32) for _ in range(12)],
        scratch_types=[
            pltpu.VMEM((80,), _i32),
            pltpu.VMEM((80,), _i32),
            pltpu.VMEM((80, 128), _f32),
            pltpu.VMEM((64, 128), _f32),
        ] + [pltpu.VMEM_SHARED((SACC, 128), _f32) for _ in range(6)],
    )


def _mm1_body(x_ref, w1_ref, w2_ref, c1_ref, c2_ref,
              o1a, o1b, o2a, o2b):
    x = x_ref[...]
    h1 = jnp.dot(x, w1_ref[...], preferred_element_type=_f32)
    h2 = jnp.dot(x, w2_ref[...], preferred_element_type=_f32)
    d1 = lax.rsqrt(c1_ref[...] + 1.0)
    d2 = lax.rsqrt(c2_ref[...] + 1.0)
    h1 = h1 * d1
    h2 = h2 * d2
    o1a[...] = h1[:, :H]
    o1b[...] = h1[:, H:]
    o2a[...] = h2[:, :H]
    o2b[...] = h2[:, H:]


def _mm1(x, w1, w2, cnt1, cnt2):
    out = jax.ShapeDtypeStruct((N, H), _f32)
    return pl.pallas_call(
        _mm1_body,
        grid=(N // MB,),
        in_specs=[
            pl.BlockSpec((MB, DIN), lambda i: (i, 0)),
            pl.BlockSpec((DIN, D), lambda i: (0, 0)),
            pl.BlockSpec((DIN, D), lambda i: (0, 0)),
            pl.BlockSpec((MB, 1), lambda i: (i, 0)),
            pl.BlockSpec((MB, 1), lambda i: (i, 0)),
        ],
        out_specs=[pl.BlockSpec((MB, H), lambda i: (i, 0))] * 4,
        out_shape=[out, out, out, out],
    )(x, w1, w2, cnt1, cnt2)


def _mid_body(s1a, s1b, s2a, s2b, g1a, g1b, g2a, g2b, c1, c2,
              bc1, bc2, mw1, mb1, mw2, mb2, wn1, wn2,
              o1a, o1b, o2a, o2b):
    d1 = lax.rsqrt(c1[...] + 1.0)
    d2 = lax.rsqrt(c2[...] + 1.0)
    s1 = jnp.concatenate([s1a[...], s1b[...]], axis=1)
    s2 = jnp.concatenate([s2a[...], s2b[...]], axis=1)
    g1 = jnp.concatenate([g1a[...], g1b[...]], axis=1)
    g2 = jnp.concatenate([g2a[...], g2b[...]], axis=1)
    h1 = jnp.maximum(d1 * (s1 + g1) + bc1[...], 0.0)
    h2 = jnp.maximum(d2 * (s2 + g2) + bc2[...], 0.0)
    hcat = jnp.concatenate([h1, h2], axis=1)
    t = jnp.maximum(jnp.dot(hcat, mw1[...], preferred_element_type=_f32)
                    + mb1[...], 0.0)
    m = jnp.dot(t, mw2[...], preferred_element_type=_f32) + mb2[...]
    n1 = jnp.dot(m, wn1[...], preferred_element_type=_f32) * d1
    n2 = jnp.dot(m, wn2[...], preferred_element_type=_f32) * d2
    o1a[...] = n1[:, :H]
    o1b[...] = n1[:, H:]
    o2a[...] = n2[:, :H]
    o2b[...] = n2[:, H:]


def _mid(scats, gs, cnt1, cnt2, bc1, bc2, mw1, mb1, mw2, mb2, wn1, wn2):
    out = jax.ShapeDtypeStruct((N, H), _f32)
    blk = pl.BlockSpec((MB, H), lambda i: (i, 0))
    full = lambda shp: pl.BlockSpec(shp, lambda i: (0, 0))
    return pl.pallas_call(
        _mid_body,
        grid=(N // MB,),
        in_specs=[blk] * 8 + [
            pl.BlockSpec((MB, 1), lambda i: (i, 0)),
            pl.BlockSpec((MB, 1), lambda i: (i, 0)),
            full((1, D)), full((1, D)),
            full((2 * D, D)), full((1, D)), full((D, D)), full((1, D)),
            full((D, D)), full((D, D)),
        ],
        out_specs=[blk] * 4,
        out_shape=[out, out, out, out],
    )(*scats, *gs, cnt1, cnt2, bc1, bc2, mw1, mb1, mw2, mb2, wn1, wn2)


def _tail_body(s1a, s1b, s2a, s2b, g1a, g1b, g2a, g2b, c1, c2,
               bc1, bc2, mw1, mb1, mw2, mb2, oa, ob):
    d1 = lax.rsqrt(c1[...] + 1.0)
    d2 = lax.rsqrt(c2[...] + 1.0)
    s1 = jnp.concatenate([s1a[...], s1b[...]], axis=1)
    s2 = jnp.concatenate([s2a[...], s2b[...]], axis=1)
    g1 = jnp.concatenate([g1a[...], g1b[...]], axis=1)
    g2 = jnp.concatenate([g2a[...], g2b[...]], axis=1)
    h1 = jnp.maximum(d1 * (s1 + g1) + bc1[...], 0.0)
    h2 = jnp.maximum(d2 * (s2 + g2) + bc2[...], 0.0)
    hcat = jnp.concatenate([h1, h2], axis=1)
    t = jnp.maximum(jnp.dot(hcat, mw1[...], preferred_element_type=_f32)
                    + mb1[...], 0.0)
    m = jnp.dot(t, mw2[...], preferred_element_type=_f32) + mb2[...]
    oa[...] = m[:, :H]
    ob[...] = m[:, H:]


def _tail(scats, gs, cnt1, cnt2, bc1, bc2, mw1, mb1, mw2, mb2):
    out = jax.ShapeDtypeStruct((N, H), _f32)
    blk = pl.BlockSpec((MB, H), lambda i: (i, 0))
    full = lambda shp: pl.BlockSpec(shp, lambda i: (0, 0))
    return pl.pallas_call(
        _tail_body,
        grid=(N // MB,),
        in_specs=[blk] * 8 + [
            pl.BlockSpec((MB, 1), lambda i: (i, 0)),
            pl.BlockSpec((MB, 1), lambda i: (i, 0)),
            full((1, D)), full((1, D)),
            full((2 * D, D)), full((1, D)), full((D, D)), full((1, D)),
        ],
        out_specs=[blk, blk],
        out_shape=[out, out],
    )(*scats, *gs, cnt1, cnt2, bc1, bc2, mw1, mb1, mw2, mb2)


def _final_body(t0a, t0b, t1a, t1b, t2a, t2b, t3a, t3b, t4a, t4b, t5a, t5b,
                c0, c1, c2, c3, c4, c5,
                w31, b31, w32, b32, wf1, bf1, wf2, bf2, out_ref):
    def mean(ta, tb, cnt):
        t = jnp.concatenate([ta[...], tb[...]], axis=1)
        return t / jnp.maximum(cnt[...], 1.0)

    x1 = mean(t0a, t0b, c0)
    x2 = mean(t1a, t1b, c1)
    xo1 = mean(t2a, t2b, c2)
    xo2 = mean(t3a, t3b, c3)
    xi1 = mean(t4a, t4b, c4)
    xi2 = mean(t5a, t5b, c5)

    def mlp3(a, b):
        hh = jnp.concatenate([a, b], axis=1)
        t = jnp.maximum(jnp.dot(hh, w31[...], preferred_element_type=_f32)
                        + b31[...], 0.0)
        return jnp.dot(t, w32[...], preferred_element_type=_f32) + b32[...]

    x_ = mlp3(x1, x2)
    xout = mlp3(xo1, xo2)
    xin = mlp3(xi1, xi2)
    xin = jnp.where(c4[...] > 0.0, xin, x_)
    xout = jnp.where(c2[...] > 0.0, xout, x_)

    hcat = jnp.concatenate([x_, xin, xout], axis=1)
    t = jnp.maximum(jnp.dot(hcat, wf1[...], preferred_element_type=_f32)
                    + bf1[...], 0.0)
    o = jnp.dot(t, wf2[...], preferred_element_type=_f32) + bf2[...]
    mx = jnp.max(o, axis=1, keepdims=True)
    e = jnp.exp(o - mx)
    lse = jnp.log(jnp.sum(e, axis=1, keepdims=True))
    out_ref[...] = o - mx - lse


def _final(tots, cnts, w31, b31, w32, b32, wf1, bf1, wf2, bf2):
    return pl.pallas_call(
        _final_body,
        out_shape=jax.ShapeDtypeStruct((S, D), _f32),
    )(*tots, *cnts, w31, b31, w32, b32, wf1, bf1, wf2, bf2)


def kernel(x, x_out, x_in,
           edge_index_1, edge_index_2, edge_index_out_1, edge_index_out_2,
           edge_index_in_1, edge_index_in_2,
           index_1, index_2, index_out_1, index_out_2, index_in_1,
           index_in_2,
           W_c11, b_c11, W_c12, b_c12, W_c21, b_c21, W_c22, b_c22,
           m1_W1, m1_b1, m1_W2, m1_b2,
           m2_W1, m2_b1, m2_W2, m2_b2,
           m3_W1, m3_b1, m3_W2, m3_b2,
           mlp_W1, mlp_b1, mlp_W2, mlp_b2):
    edges = (edge_index_1, edge_index_2, edge_index_out_1, edge_index_out_2,
             edge_index_in_1, edge_index_in_2)
    idxs = (index_1, index_2, index_out_1, index_out_2, index_in_1,
            index_in_2)

    srcs = [e[0] for e in edges]
    dsts = [e[1] for e in edges]
    idx_pad = [jnp.pad(ix, (0, NP - N), constant_values=S) for ix in idxs]
    zeros = jnp.zeros((625, 128), _f32)
    zeros1 = jnp.zeros((NP,), _f32)

    couts = _counts_kernel()(*dsts, *idx_pad, zeros1)
    sums = _csum(couts)
    ecnt = [sums[j][:N].reshape(N, 1) for j in range(6)]
    scnt = [sums[6 + j][:S].reshape(S, 1) for j in range(6)]

    b_c11r = b_c11.reshape(1, D)
    b_c12r = b_c12.reshape(1, D)
    b_c21r = b_c21.reshape(1, D)
    b_c22r = b_c22.reshape(1, D)
    m1_b1r = m1_b1.reshape(1, D)
    m1_b2r = m1_b2.reshape(1, D)
    m2_b1r = m2_b1.reshape(1, D)
    m2_b2r = m2_b2.reshape(1, D)

    def branch(xb, e1, e2):
        c1, c2 = ecnt[e1], ecnt[e2]
        g1a, g1b, g2a, g2b = _mm1(xb, W_c11, W_c12, c1, c2)
        s1a, s1b = _gcn_kernel()(srcs[e1], dsts[e1], g1a, g1b, zeros)
        s2a, s2b = _gcn_kernel()(srcs[e2], dsts[e2], g2a, g2b, zeros)
        n1a, n1b, n2a, n2b = _mid(
            (s1a, s1b, s2a, s2b), (g1a, g1b, g2a, g2b), c1, c2,
            b_c11r, b_c12r, m1_W1, m1_b1r, m1_W2, m1_b2r, W_c21, W_c22)
        s1a, s1b = _gcn_kernel()(srcs[e1], dsts[e1], n1a, n1b, zeros)
        s2a, s2b = _gcn_kernel()(srcs[e2], dsts[e2], n2a, n2b, zeros)
        return _tail((s1a, s1b, s2a, s2b), (n1a, n1b, n2a, n2b), c1, c2,
                     b_c21r, b_c22r, m2_W1, m2_b1r, m2_W2, m2_b2r)

    hoa, hob = branch(x, 0, 1)
    houta, houtb = branch(x_out, 2, 3)
    hina, hinb = branch(x_in, 4, 5)

    pad2 = lambda a: jnp.pad(a, ((0, NP - N), (0, 0)))
    tots = _ssum_kernel()(
        pad2(hoa), pad2(hob), pad2(houta), pad2(houtb),
        pad2(hina), pad2(hinb),
        idx_pad[0], idx_pad[1], idx_pad[2], idx_pad[3], idx_pad[4],
        idx_pad[5], zeros)

    return _final(tots, scnt,
                  m3_W1, m3_b1.reshape(1, D), m3_W2, m3_b2.reshape(1, D),
                  mlp_W1, mlp_b1.reshape(1, 2 * D),
                  mlp_W2, mlp_b2.reshape(1, D))


# speedup vs baseline: 1.3740x; 1.0162x over previous
import functools
import jax
import jax.numpy as jnp
from jax import lax
from jax.experimental import pallas as pl
from jax.experimental.pallas import tpu as pltpu
from jax.experimental.pallas import tpu_sc as plsc

N = 10000
E = 160000
S = 1024
DIN = 2050
D = 256
H = 128
NP = 10240
TPT = NP // 16
EPT = E // 16
G = 128
NCH = EPT // G
GT = EPT - NCH * G
MB = 400
SACC = 1040

@functools.lru_cache(maxsize=None)
def _mesh():
    return plsc.VectorSubcoreMesh(core_axis_name="c", subcore_axis_name="s")

_f32 = jnp.float32
_i32 = jnp.int32


SP = 1152


def _counts_body(d0, d1, d2, d3, d4, d5, x0, x1, x2, x3, x4, x5, z1,
                 oe0, oe1, oe2, oe3, oe4, oe5,
                 os0, os1, os2, os3, os4, os5,
                 ebuf, xbuf, priv, privs):
    c = lax.axis_index("c")
    s = lax.axis_index("s")
    ones = jnp.full((16,), 1.0, _f32)
    dsets = ((d0, d1, d2), (d3, d4, d5))
    xsets = ((x0, x1, x2), (x3, x4, x5))
    eouts = ((oe0, oe1, oe2), (oe3, oe4, oe5))
    souts = ((os0, os1, os2), (os3, os4, os5))

    for k in range(3):
        for core in range(2):
            @pl.when(c == core)
            def _(dref=dsets[core][k], oref=eouts[core][k]):
                pltpu.sync_copy(z1.at[pl.ds(0, NP)], priv)
                pltpu.sync_copy(dref.at[pl.ds(s * EPT, EPT)], ebuf)

                def it(i, carry):
                    for u in range(5):
                        ii = ebuf[pl.ds(i * 80 + u * 16, 16)]
                        plsc.addupdate_scatter(priv, [ii], ones)
                    return carry

                lax.fori_loop(0, EPT // 80, it, 0)
                pltpu.sync_copy(priv, oref.at[pl.ds(s * NP, NP)])

    for k in range(3):
        for core in range(2):
            @pl.when(c == core)
            def _(xref=xsets[core][k], oref=souts[core][k]):
                pltpu.sync_copy(z1.at[pl.ds(0, SP)], privs)
                pltpu.sync_copy(xref.at[pl.ds(s * TPT, TPT)], xbuf)

                for i in range(TPT // 80):
                    for u in range(5):
                        ii = xbuf[pl.ds(i * 80 + u * 16, 16)]
                        plsc.addupdate_scatter(privs, [ii], ones)
                pltpu.sync_copy(privs, oref.at[pl.ds(s * SP, SP)])


@functools.lru_cache(maxsize=None)
def _counts_kernel():
    return pl.kernel(
        _counts_body, mesh=_mesh(), name="sc_counts",
        compiler_params=pltpu.CompilerParams(needs_layout_passes=False),
        out_type=[jax.ShapeDtypeStruct((16 * NP,), _f32) for _ in range(6)]
        + [jax.ShapeDtypeStruct((16 * SP,), _f32) for _ in range(6)],
        scratch_types=[
            pltpu.VMEM((EPT,), _i32),
            pltpu.VMEM((TPT,), _i32),
            pltpu.VMEM((NP,), _f32),
            pltpu.VMEM((SP,), _f32),
        ],
    )


def _csum_body(*refs):
    ins = refs[:12]
    outs = refs[12:]
    for i in range(12):
        outs[i][...] = jnp.sum(ins[i][...], axis=0)


def _csum(couts):
    ins = [cc.reshape(16, NP) for cc in couts[:6]]
    ins += [cc.reshape(16, SP) for cc in couts[6:]]
    return pl.pallas_call(
        _csum_body,
        out_shape=[jax.ShapeDtypeStruct((NP,), _f32) for _ in range(6)]
        + [jax.ShapeDtypeStruct((SP,), _f32) for _ in range(6)],
    )(*ins)


def _gcn_body(src, dst, ha, hb, z, oa, ob,
              sb0, sb1, sb2, sb3, db0, db1, db2, db3, st, dt,
              r0, r1, wb, acc,
              gs0, gs1, ss0, ss1, is0, is1, is2, is3):
    c = lax.axis_index("c")
    s = lax.axis_index("s")
    pltpu.sync_copy(z.at[pl.ds(0, 104)], wb)
    for j in range(6):
        pltpu.sync_copy(wb, acc.at[pl.ds(s * 624 + j * 104, 104)])

    @pl.when(s == 15)
    def _():
        pltpu.sync_copy(wb.at[pl.ds(0, 16)], acc.at[pl.ds(9984, 16)])

    plsc.subcore_barrier()
    sbufs = (sb0, sb1, sb2, sb3)
    dbufs = (db0, db1, db2, db3)
    rws = (r0, r1)
    gsems = (gs0, gs1)
    ssems = (ss0, ss1)
    isems = (is0, is1, is2, is3)
    base = s * EPT

    def ld_idx(i, j):
        off = base + i * G
        pltpu.async_copy(src.at[pl.ds(off, G)], sbufs[j], isems[j])
        pltpu.async_copy(dst.at[pl.ds(off, G)], dbufs[j], isems[j])

    def wait_idx(i, j):
        off = base + i * G
        pltpu.make_async_copy(src.at[pl.ds(off, G)], sbufs[j],
                              isems[j]).wait()
        pltpu.make_async_copy(dst.at[pl.ds(off, G)], dbufs[j],
                              isems[j]).wait()

    def run(href):
        pltpu.sync_copy(src.at[pl.ds(base, G)], sb0)
        pltpu.sync_copy(dst.at[pl.ds(base, G)], db0)
        ld_idx(1, 1)
        ld_idx(2, 2)
        pltpu.async_copy(href.at[sb0], r0, gs0)

        def sub(i, b, j):
            nb = 1 - b
            pltpu.make_async_copy(href.at[sbufs[j]], rws[b],
                                  gsems[b]).wait()
            pltpu.async_copy(rws[b], acc.at[dbufs[j]], ssems[b], add=True)

            @pl.when(i + 1 < NCH)
            def _():
                wait_idx(i + 1, (j + 1) % 4)

            @pl.when(i >= 1)
            def _():
                pltpu.make_async_copy(
                    rws[nb], acc.at[dbufs[(j + 3) % 4]],
                    ssems[nb]).wait()

            @pl.when(i + 1 < NCH)
            def _():
                pltpu.async_copy(href.at[sbufs[(j + 1) % 4]], rws[nb],
                                 gsems[nb])

            @pl.when(i + 3 < NCH)
            def _():
                ld_idx(i + 3, (j + 3) % 4)

        def outer(k, carry):
            for u in range(4):
                sub(4 * k + u, u % 2, u)
            return carry

        lax.fori_loop(0, NCH // 4, outer, 0)
        for i in range(NCH - NCH % 4, NCH):
            sub(i, i % 2, i % 4)
        pltpu.make_async_copy(rws[(NCH - 1) % 2],
                              acc.at[dbufs[(NCH - 1) % 4]],
                              ssems[(NCH - 1) % 2]).wait()
        pltpu.sync_copy(src.at[pl.ds(base + NCH * G, GT)], st)
        pltpu.sync_copy(dst.at[pl.ds(base + NCH * G, GT)], dt)
        pltpu.async_copy(href.at[st], r0.at[pl.ds(0, GT)], gs0).wait()
        pltpu.sync_copy(r0.at[pl.ds(0, GT)], acc.at[dt], add=True)

    for core, href in ((0, ha), (1, hb)):
        @pl.when(c == core)
        def _(href=href):
            run(href)

    plsc.subcore_barrier()
    for j in range(6):
        pltpu.sync_copy(acc.at[pl.ds(s * 624 + j * 104, 104)], wb)
        pl.when(c == 0)(lambda j=j: pltpu.sync_copy(
            wb, oa.at[pl.ds(s * 624 + j * 104, 104)]))
        pl.when(c == 1)(lambda j=j: pltpu.sync_copy(
            wb, ob.at[pl.ds(s * 624 + j * 104, 104)]))

    @pl.when(s == 15)
    def _():
        pltpu.sync_copy(acc.at[pl.ds(9984, 16)], r0.at[pl.ds(0, 16)])
        pl.when(c == 0)(lambda: pltpu.sync_copy(
            r0.at[pl.ds(0, 16)], oa.at[pl.ds(9984, 16)]))
        pl.when(c == 1)(lambda: pltpu.sync_copy(
            r0.at[pl.ds(0, 16)], ob.at[pl.ds(9984, 16)]))


@functools.lru_cache(maxsize=None)
def _gcn_kernel():
    return pl.kernel(
        _gcn_body, mesh=_mesh(), name="sc_gcn",
        compiler_params=pltpu.CompilerParams(needs_layout_passes=False),
        out_type=[jax.ShapeDtypeStruct((N, H), _f32),
                  jax.ShapeDtypeStruct((N, H), _f32)],
        scratch_types=(
            [pltpu.VMEM((G,), _i32) for _ in range(8)]
            + [pltpu.VMEM((GT,), _i32) for _ in range(2)]
            + [
                pltpu.VMEM((G, 128), _f32),
                pltpu.VMEM((G, 128), _f32),
                pltpu.VMEM((104, 128), _f32),
                pltpu.VMEM_SHARED((N, 128), _f32),
            ]
            + [pltpu.SemaphoreType.DMA for _ in range(8)]
        ),
    )


def _ssum_body(h0a, h0b, h1a, h1b, h2a, h2b,
               i0, i1, i2, i3, i4, i5, z,
               o0a, o0b, o1a, o1b, o2a, o2b,
               o3a, o3b, o4a, o4b, o5a, o5b,
               iA0, iA1, iB0, iB1, rw0, rw1, wb,
               a0, a1, a2, a3, a4, a5,
               is0, is1, rs0, rs1, sa0, sa1, sb0, sb1):
    ibufA = (iA0, iA1)
    ibufB = (iB0, iB1)
    rows = (rw0, rw1)
    isem = (is0, is1)
    rsem = (rs0, rs1)
    sA = (sa0, sa1)
    sB = (sb0, sb1)
    c = lax.axis_index("c")
    s = lax.axis_index("s")
    accs = (a0, a1, a2, a3, a4, a5)
    pltpu.sync_copy(z.at[pl.ds(0, 64)], wb)
    for a in accs:
        pltpu.sync_copy(wb, a.at[pl.ds(s * 64, 64)])
    plsc.subcore_barrier()

    hs = ((h0a, h0b), (h1a, h1b), (h2a, h2b))
    idxs = ((i0, i1), (i2, i3), (i4, i5))
    CH = 128
    NCS = TPT // CH
    NT = 3 * NCS

    def chunk_refs(t, core):
        p, k = divmod(t, NCS)
        return (hs[p][core], idxs[p][0], idxs[p][1],
                accs[2 * p], accs[2 * p + 1], k)

    for core in range(2):
        @pl.when(c == core)
        def _(core=core):
            def loads(t, b):
                href, iA, iB, _, _, k = chunk_refs(t, core)
                off = s * TPT + k * CH
                pltpu.async_copy(iA.at[pl.ds(off, CH)], ibufA[b], isem[b])
                pltpu.async_copy(iB.at[pl.ds(off, CH)], ibufB[b], isem[b])
                pltpu.async_copy(href.at[pl.ds(off, CH)], rows[b], rsem[b])

            def wait_loads(t, b):
                href, iA, iB, _, _, k = chunk_refs(t, core)
                off = s * TPT + k * CH
                pltpu.make_async_copy(iA.at[pl.ds(off, CH)], ibufA[b],
                                      isem[b]).wait()
                pltpu.make_async_copy(iB.at[pl.ds(off, CH)], ibufB[b],
                                      isem[b]).wait()
                pltpu.make_async_copy(href.at[pl.ds(off, CH)], rows[b],
                                      rsem[b]).wait()

            def wait_scat(t, b):
                _, _, _, accA, accB, _ = chunk_refs(t, core)
                pltpu.make_async_copy(rows[b], accA.at[ibufA[b]],
                                      sA[b]).wait()
                pltpu.make_async_copy(rows[b], accB.at[ibufB[b]],
                                      sB[b]).wait()

            loads(0, 0)
            for t in range(NT):
                b = t % 2
                nb = 1 - b
                wait_loads(t, b)
                _, _, _, accA, accB, _ = chunk_refs(t, core)
                pltpu.async_copy(rows[b], accA.at[ibufA[b]], sA[b],
                                 add=True)
                pltpu.async_copy(rows[b], accB.at[ibufB[b]], sB[b],
                                 add=True)
                if t + 1 < NT:
                    if t >= 1:
                        wait_scat(t - 1, nb)
                    loads(t + 1, nb)
            wait_scat(NT - 2, (NT - 2) % 2)
            wait_scat(NT - 1, (NT - 1) % 2)

    plsc.subcore_barrier()
    outs = ((o0a, o0b), (o1a, o1b), (o2a, o2b),
            (o3a, o3b), (o4a, o4b), (o5a, o5b))
    for k in range(6):
        pltpu.sync_copy(accs[k].at[pl.ds(s * 64, 64)], wb)
        for core in range(2):
            pl.when(c == core)(
                lambda oref=outs[k][core]: pltpu.sync_copy(
                    wb, oref.at[pl.ds(s * 64, 64)]))


@functools.lru_cache(maxsize=None)
def _ssum_kernel():
    return pl.kernel(
        _ssum_body, mesh=_mesh(), name="sc_ssum",
        compiler_params=pltpu.CompilerParams(needs_layout_passes=False),
        out_type=[jax.ShapeDtypeStruct((S, H), _f32) for _ in range(12)],
        scratch_types=[
            pltpu.VMEM((128,), _i32),
            pltpu.VMEM((128,), _i32),
            pltpu.VMEM((128,), _i32),
            pltpu.VMEM((128,), _i32),
            pltpu.VMEM((128, 128), _f32),
            pltpu.VMEM((128, 128), _f32),
            pltpu.VMEM((64, 128), _f32),
        ] + [pltpu.VMEM_SHARED((SACC, 128), _f32) for _ in range(6)]
        + [pltpu.SemaphoreType.DMA for _ in range(8)],
    )


def _mm1_body(x_ref, w1_ref, w2_ref, c1_ref, c2_ref,
              o1a, o1b, o2a, o2b):
    x = x_ref[...]
    h1 = jnp.dot(x, w1_ref[...], preferred_element_type=_f32)
    h2 = jnp.dot(x, w2_ref[...], preferred_element_type=_f32)
    d1 = lax.rsqrt(c1_ref[...] + 1.0)
    d2 = lax.rsqrt(c2_ref[...] + 1.0)
    h1 = h1 * d1
    h2 = h2 * d2
    o1a[...] = h1[:, :H]
    o1b[...] = h1[:, H:]
    o2a[...] = h2[:, :H]
    o2b[...] = h2[:, H:]


def _mm1(x, w1, w2, cnt1, cnt2):
    out = jax.ShapeDtypeStruct((N, H), _f32)
    return pl.pallas_call(
        _mm1_body,
        grid=(N // MB,),
        in_specs=[
            pl.BlockSpec((MB, DIN), lambda i: (i, 0)),
            pl.BlockSpec((DIN, D), lambda i: (0, 0)),
            pl.BlockSpec((DIN, D), lambda i: (0, 0)),
            pl.BlockSpec((MB, 1), lambda i: (i, 0)),
            pl.BlockSpec((MB, 1), lambda i: (i, 0)),
        ],
        out_specs=[pl.BlockSpec((MB, H), lambda i: (i, 0))] * 4,
        out_shape=[out, out, out, out],
    )(x, w1, w2, cnt1, cnt2)


def _mid_body(s1a, s1b, s2a, s2b, g1a, g1b, g2a, g2b, c1, c2,
              bc1, bc2, mw1, mb1, mw2, mb2, wn1, wn2,
              o1a, o1b, o2a, o2b):
    d1 = lax.rsqrt(c1[...] + 1.0)
    d2 = lax.rsqrt(c2[...] + 1.0)
    s1 = jnp.concatenate([s1a[...], s1b[...]], axis=1)
    s2 = jnp.concatenate([s2a[...], s2b[...]], axis=1)
    g1 = jnp.concatenate([g1a[...], g1b[...]], axis=1)
    g2 = jnp.concatenate([g2a[...], g2b[...]], axis=1)
    h1 = jnp.maximum(d1 * (s1 + g1) + bc1[...], 0.0)
    h2 = jnp.maximum(d2 * (s2 + g2) + bc2[...], 0.0)
    hcat = jnp.concatenate([h1, h2], axis=1)
    t = jnp.maximum(jnp.dot(hcat, mw1[...], preferred_element_type=_f32)
                    + mb1[...], 0.0)
    m = jnp.dot(t, mw2[...], preferred_element_type=_f32) + mb2[...]
    n1 = jnp.dot(m, wn1[...], preferred_element_type=_f32) * d1
    n2 = jnp.dot(m, wn2[...], preferred_element_type=_f32) * d2
    o1a[...] = n1[:, :H]
    o1b[...] = n1[:, H:]
    o2a[...] = n2[:, :H]
    o2b[...] = n2[:, H:]


def _mid(scats, gs, cnt1, cnt2, bc1, bc2, mw1, mb1, mw2, mb2, wn1, wn2):
    out = jax.ShapeDtypeStruct((N, H), _f32)
    blk = pl.BlockSpec((MB, H), lambda i: (i, 0))
    full = lambda shp: pl.BlockSpec(shp, lambda i: (0, 0))
    return pl.pallas_call(
        _mid_body,
        grid=(N // MB,),
        in_specs=[blk] * 8 + [
            pl.BlockSpec((MB, 1), lambda i: (i, 0)),
            pl.BlockSpec((MB, 1), lambda i: (i, 0)),
            full((1, D)), full((1, D)),
            full((2 * D, D)), full((1, D)), full((D, D)), full((1, D)),
            full((D, D)), full((D, D)),
        ],
        out_specs=[blk] * 4,
        out_shape=[out, out, out, out],
    )(*scats, *gs, cnt1, cnt2, bc1, bc2, mw1, mb1, mw2, mb2, wn1, wn2)


def _tail_body(s1a, s1b, s2a, s2b, g1a, g1b, g2a, g2b, c1, c2,
               bc1, bc2, mw1, mb1, mw2, mb2, oa, ob):
    d1 = lax.rsqrt(c1[...] + 1.0)
    d2 = lax.rsqrt(c2[...] + 1.0)
    s1 = jnp.concatenate([s1a[...], s1b[...]], axis=1)
    s2 = jnp.concatenate([s2a[...], s2b[...]], axis=1)
    g1 = jnp.concatenate([g1a[...], g1b[...]], axis=1)
    g2 = jnp.concatenate([g2a[...], g2b[...]], axis=1)
    h1 = jnp.maximum(d1 * (s1 + g1) + bc1[...], 0.0)
    h2 = jnp.maximum(d2 * (s2 + g2) + bc2[...], 0.0)
    hcat = jnp.concatenate([h1, h2], axis=1)
    t = jnp.maximum(jnp.dot(hcat, mw1[...], preferred_element_type=_f32)
                    + mb1[...], 0.0)
    m = jnp.dot(t, mw2[...], preferred_element_type=_f32) + mb2[...]
    oa[...] = m[:, :H]
    ob[...] = m[:, H:]


def _tail(scats, gs, cnt1, cnt2, bc1, bc2, mw1, mb1, mw2, mb2):
    out = jax.ShapeDtypeStruct((N, H), _f32)
    blk = pl.BlockSpec((MB, H), lambda i: (i, 0))
    full = lambda shp: pl.BlockSpec(shp, lambda i: (0, 0))
    return pl.pallas_call(
        _tail_body,
        grid=(N // MB,),
        in_specs=[blk] * 8 + [
            pl.BlockSpec((MB, 1), lambda i: (i, 0)),
            pl.BlockSpec((MB, 1), lambda i: (i, 0)),
            full((1, D)), full((1, D)),
            full((2 * D, D)), full((1, D)), full((D, D)), full((1, D)),
        ],
        out_specs=[blk, blk],
        out_shape=[out, out],
    )(*scats, *gs, cnt1, cnt2, bc1, bc2, mw1, mb1, mw2, mb2)


def _final_body(t0a, t0b, t1a, t1b, t2a, t2b, t3a, t3b, t4a, t4b, t5a, t5b,
                c0, c1, c2, c3, c4, c5,
                w31, b31, w32, b32, wf1, bf1, wf2, bf2, out_ref):
    def mean(ta, tb, cnt):
        t = jnp.concatenate([ta[...], tb[...]], axis=1)
        return t / jnp.maximum(cnt[...], 1.0)

    x1 = mean(t0a, t0b, c0)
    x2 = mean(t1a, t1b, c1)
    xo1 = mean(t2a, t2b, c2)
    xo2 = mean(t3a, t3b, c3)
    xi1 = mean(t4a, t4b, c4)
    xi2 = mean(t5a, t5b, c5)

    def mlp3(a, b):
        hh = jnp.concatenate([a, b], axis=1)
        t = jnp.maximum(jnp.dot(hh, w31[...], preferred_element_type=_f32)
                        + b31[...], 0.0)
        return jnp.dot(t, w32[...], preferred_element_type=_f32) + b32[...]

    x_ = mlp3(x1, x2)
    xout = mlp3(xo1, xo2)
    xin = mlp3(xi1, xi2)
    xin = jnp.where(c4[...] > 0.0, xin, x_)
    xout = jnp.where(c2[...] > 0.0, xout, x_)

    hcat = jnp.concatenate([x_, xin, xout], axis=1)
    t = jnp.maximum(jnp.dot(hcat, wf1[...], preferred_element_type=_f32)
                    + bf1[...], 0.0)
    o = jnp.dot(t, wf2[...], preferred_element_type=_f32) + bf2[...]
    mx = jnp.max(o, axis=1, keepdims=True)
    e = jnp.exp(o - mx)
    lse = jnp.log(jnp.sum(e, axis=1, keepdims=True))
    out_ref[...] = o - mx - lse


def _final(tots, cnts, w31, b31, w32, b32, wf1, bf1, wf2, bf2):
    return pl.pallas_call(
        _final_body,
        out_shape=jax.ShapeDtypeStruct((S, D), _f32),
    )(*tots, *cnts, w31, b31, w32, b32, wf1, bf1, wf2, bf2)


def kernel(x, x_out, x_in,
           edge_index_1, edge_index_2, edge_index_out_1, edge_index_out_2,
           edge_index_in_1, edge_index_in_2,
           index_1, index_2, index_out_1, index_out_2, index_in_1,
           index_in_2,
           W_c11, b_c11, W_c12, b_c12, W_c21, b_c21, W_c22, b_c22,
           m1_W1, m1_b1, m1_W2, m1_b2,
           m2_W1, m2_b1, m2_W2, m2_b2,
           m3_W1, m3_b1, m3_W2, m3_b2,
           mlp_W1, mlp_b1, mlp_W2, mlp_b2):
    edges = (edge_index_1, edge_index_2, edge_index_out_1, edge_index_out_2,
             edge_index_in_1, edge_index_in_2)
    idxs = (index_1, index_2, index_out_1, index_out_2, index_in_1,
            index_in_2)

    srcs = [e[0] for e in edges]
    dsts = [e[1] for e in edges]
    idx_pad = [jnp.pad(ix, (0, NP - N), constant_values=S) for ix in idxs]
    zeros = jnp.zeros((625, 128), _f32)
    zeros1 = jnp.zeros((NP,), _f32)

    couts = _counts_kernel()(*dsts, *idx_pad, zeros1)
    sums = _csum(couts)
    ecnt = [sums[j][:N].reshape(N, 1) for j in range(6)]
    scnt = [sums[6 + j][:S].reshape(S, 1) for j in range(6)]

    b_c11r = b_c11.reshape(1, D)
    b_c12r = b_c12.reshape(1, D)
    b_c21r = b_c21.reshape(1, D)
    b_c22r = b_c22.reshape(1, D)
    m1_b1r = m1_b1.reshape(1, D)
    m1_b2r = m1_b2.reshape(1, D)
    m2_b1r = m2_b1.reshape(1, D)
    m2_b2r = m2_b2.reshape(1, D)

    def branch(xb, e1, e2):
        c1, c2 = ecnt[e1], ecnt[e2]
        g1a, g1b, g2a, g2b = _mm1(xb, W_c11, W_c12, c1, c2)
        s1a, s1b = _gcn_kernel()(srcs[e1], dsts[e1], g1a, g1b, zeros)
        s2a, s2b = _gcn_kernel()(srcs[e2], dsts[e2], g2a, g2b, zeros)
        n1a, n1b, n2a, n2b = _mid(
            (s1a, s1b, s2a, s2b), (g1a, g1b, g2a, g2b), c1, c2,
            b_c11r, b_c12r, m1_W1, m1_b1r, m1_W2, m1_b2r, W_c21, W_c22)
        s1a, s1b = _gcn_kernel()(srcs[e1], dsts[e1], n1a, n1b, zeros)
        s2a, s2b = _gcn_kernel()(srcs[e2], dsts[e2], n2a, n2b, zeros)
        return _tail((s1a, s1b, s2a, s2b), (n1a, n1b, n2a, n2b), c1, c2,
                     b_c21r, b_c22r, m2_W1, m2_b1r, m2_W2, m2_b2r)

    hoa, hob = branch(x, 0, 1)
    houta, houtb = branch(x_out, 2, 3)
    hina, hinb = branch(x_in, 4, 5)

    pad2 = lambda a: jnp.pad(a, ((0, NP - N), (0, 0)))
    tots = _ssum_kernel()(
        pad2(hoa), pad2(hob), pad2(houta), pad2(houtb),
        pad2(hina), pad2(hinb),
        idx_pad[0], idx_pad[1], idx_pad[2], idx_pad[3], idx_pad[4],
        idx_pad[5], zeros)

    return _final(tots, scnt,
                  m3_W1, m3_b1.reshape(1, D), m3_W2, m3_b2.reshape(1, D),
                  mlp_W1, mlp_b1.reshape(1, 2 * D),
                  mlp_W2, mlp_b2.reshape(1, D))
